# baseline (speedup 1.0000x reference)
import functools

import jax
import jax.numpy as jnp
from jax import lax
from jax.experimental import pallas as pl
from jax.experimental.pallas import tpu as pltpu
from jax.experimental.pallas import tpu_sc as plsc

_SCALE = 10.0
_NC = 2
_NS = 16
_CH = 128
_NB1 = 3
_NBUF = 4


@functools.lru_cache(maxsize=None)
def _make_relayout(V, D):
  dh = D // _NC
  n_full = V // _CH
  tail = V - n_full * _CH
  t0 = n_full * _CH
  bpt = (n_full + _NS - 1) // _NS
  mesh = plsc.VectorSubcoreMesh(core_axis_name="c", subcore_axis_name="s")

  @functools.partial(
      pl.kernel,
      mesh=mesh,
      out_type=jax.ShapeDtypeStruct(((_NC * t0 + 2 * tail) * dh // _CH, _CH), jnp.float32),
      scratch_types=[
          [pltpu.VMEM((dh, _CH), jnp.float32)] * _NB1,
          [pltpu.VMEM((dh, _CH), jnp.float32)] * _NB1,
          pltpu.VMEM((dh, _CH), jnp.float32),
          [pltpu.SemaphoreType.DMA] * _NB1,
          [pltpu.SemaphoreType.DMA] * _NB1,
      ],
      compiler_params=pltpu.CompilerParams(
          use_tc_tiling_on_sc=True, needs_layout_passes=False),
  )
  def relayout(wt_hbm, tail_hbm, scr_hbm, tb, rb, tt, t_sems, r_sems):
    core = lax.axis_index("c")
    tid = lax.axis_index("s")
    lanes = lax.iota(jnp.int32, 16)
    dbase = core * dh
    srow0 = core * (t0 * dh // _CH)

    def read(b, si):
      for j in range(_NC):
        pltpu.async_copy(
            wt_hbm.at[pl.ds(dbase + j * 8, 8), pl.ds(b * _CH, _CH)],
            tb[si].at[pl.ds(j * 8, 8)], t_sems[si])

    def wait_read(si):
      for j in range(_NC):
        pltpu.make_async_copy(
            wt_hbm.at[pl.ds(dbase + j * 8, 8), pl.ds(0, _CH)],
            tb[si].at[pl.ds(j * 8, 8)], t_sems[si]).wait()

    def write(b, si):
      pltpu.async_copy(
          rb[si], scr_hbm.at[pl.ds(srow0 + b * dh, dh)], r_sems[si])

    def wait_write(si):
      pltpu.make_async_copy(
          rb[si], scr_hbm.at[pl.ds(srow0, dh)], r_sems[si]).wait()

    lo = tid * bpt
    n_my = jnp.maximum(jnp.minimum(lo + bpt, n_full) - lo, 0)

    for si in range(_NB1):
      @pl.when(si < n_my)
      def _():
        read(lo + si, si)

    i0 = lax.shift_right_logical(lanes, 3)
    i1 = lanes & 7
    zeros = jnp.zeros((16,), jnp.int32)

    def transpose_block(si):
      @plsc.parallel_loop(0, dh, unroll=4)
      def _(r8):
        base = r8 * 8
        for c in range(8):
          v = plsc.load_gather(tb[si], [lanes, zeros + (base + c)])
          rb[si][r8, pl.ds(c * 16, 16)] = v

    def outer(k0, carry):
      for si in range(_NB1):
        k = k0 + si
        @pl.when(k < n_my)
        def _():
          b = lo + k
          wait_read(si)
          @pl.when(k >= _NB1)
          def _():
            wait_write(si)
          transpose_block(si)
          write(b, si)
          @pl.when(k + _NB1 < n_my)
          def _():
            read(b + _NB1, si)
      return carry

    lax.fori_loop(0, (bpt + _NB1 - 1) // _NB1,
                  lambda i, cr: outer(i * _NB1, cr), 0)

    for si in range(_NB1):
      @pl.when(n_my > si)
      def _():
        wait_write(si)

    if tail:
      @pl.when((tid == 0) & (core == 0))
      def _():
        pltpu.sync_copy(tail_hbm, tt)
        pltpu.sync_copy(
            tt, scr_hbm.at[pl.ds(_NC * t0 * dh // _CH, 2 * tail * dh // _CH)])

  return relayout


@functools.lru_cache(maxsize=None)
def _make_lookup(V, D, S0, S1):
  B = S0 * S1
  n_st = S0 // _CH
  n_dt = D // 8
  dh = D // _NC
  n_ch_tot = B // _CH
  n_ch = n_ch_tot // _NS
  n_tiles = n_ch_tot * n_dt
  assert n_ch % _NBUF == 0
  mesh = plsc.VectorSubcoreMesh(core_axis_name="c", subcore_axis_name="s")

  @functools.partial(
      pl.kernel,
      mesh=mesh,
      out_type=jax.ShapeDtypeStruct((n_tiles, 8, _CH), jnp.float32),
      scratch_types=[
          pltpu.VMEM((n_ch, _CH), jnp.int32),
          [pltpu.VMEM((_CH,), jnp.int32)] * _NBUF,
          [pltpu.VMEM((_CH, dh), jnp.float32)] * _NBUF,
          [pltpu.VMEM((_NC, 8, _CH), jnp.float32)] * _NBUF,
          [pltpu.SemaphoreType.DMA] * _NBUF,
          [pltpu.SemaphoreType.DMA] * _NBUF,
      ],
      compiler_params=pltpu.CompilerParams(
          use_tc_tiling_on_sc=False, needs_layout_passes=False),
  )
  def lookup(scr_hbm, idx_hbm, out_hbm, idx_v, p_v, g_b, o_b,
             in_sems, out_sems):
    core = lax.axis_index("c")
    tid = lax.axis_index("s")
    lanes = lax.iota(jnp.int32, 16)
    t0 = (V // _CH) * _CH
    tail = V - t0
    off_lo = core * t0
    off_hi = t0 + core * tail
    ch_base = tid * n_ch
    pltpu.sync_copy(idx_hbm.at[pl.ds(ch_base, n_ch)], idx_v)

    def gather(cl, bi):
      @plsc.parallel_loop(0, _CH // 16, unroll=4)
      def _(g):
        rv = idx_v[cl, pl.ds(g * 16, 16)]
        p_v[bi][pl.ds(g * 16, 16)] = rv + jnp.where(rv < t0, off_lo, off_hi)
      return pltpu.async_copy(scr_hbm.at[p_v[bi]], g_b[bi], in_sems[bi])

    def wait_gather(bi):
      pltpu.make_async_copy(scr_hbm.at[p_v[bi]], g_b[bi], in_sems[bi]).wait()

    def out_tile_base(cl):
      c = ch_base + cl
      s1 = lax.div(c, n_st)
      st = lax.rem(c, n_st)
      return (s1 * n_dt + core * _NC) * n_st + st

    def write(cl, bi):
      base = out_tile_base(cl)
      for j in range(_NC):
        pltpu.async_copy(
            o_b[bi].at[pl.ds(j, 1)],
            out_hbm.at[pl.ds(base + j * n_st, 1)], out_sems[bi])

    def wait_write(cl, bi):
      base = out_tile_base(cl)
      for j in range(_NC):
        pltpu.make_async_copy(
            o_b[bi].at[pl.ds(j, 1)],
            out_hbm.at[pl.ds(base + j * n_st, 1)], out_sems[bi]).wait()

    for bi in range(_NBUF):
      gather(bi, bi)

    def outer(c0, carry):
      for bi in range(_NBUF):
        cl = c0 + bi
        wait_gather(bi)
        @pl.when(cl >= _NBUF)
        def _():
          wait_write(cl - _NBUF, bi)

        @plsc.parallel_loop(0, _CH // 16, unroll=2)
        def _(g):
          rows = g * 16 + lanes
          for d in range(dh):
            v = plsc.load_gather(g_b[bi], [rows, jnp.full((16,), d, jnp.int32)])
            o_b[bi][d // 8, d % 8, pl.ds(g * 16, 16)] = v * _SCALE

        write(cl, bi)
        @pl.when(cl + _NBUF < n_ch)
        def _():
          gather(cl + _NBUF, bi)
      return carry

    lax.fori_loop(0, n_ch // _NBUF, lambda i, cr: outer(i * _NBUF, cr), 0)

    for bi in range(_NBUF):
      wait_write(n_ch - _NBUF + bi, bi)

  return lookup


def kernel(x, weight):
  S0, S1 = x.shape
  V, D = weight.shape
  B = S0 * S1
  dh = D // _NC
  t0 = (V // _CH) * _CH
  idx = x.T.reshape(B // _CH, _CH).astype(jnp.int32)
  tail_rows = jnp.concatenate(
      [weight[t0:, :dh], weight[t0:, dh:]], axis=0).reshape(dh, _CH)
  scratch = _make_relayout(V, D)(weight.T, tail_rows)
  scratch = scratch.reshape(-1, dh)
  out_t = _make_lookup(V, D, S0, S1)(scratch, idx)
  n_st = S0 // _CH
  n_dt = D // 8
  out = out_t.reshape(S1, n_dt, n_st, 8, _CH).transpose(2, 4, 0, 1, 3)
  return out.reshape(S0, S1, D)


# speedup vs baseline: 1.8734x; 1.8734x over previous
import functools

import jax
import jax.numpy as jnp
from jax import lax
from jax.experimental import pallas as pl
from jax.experimental.pallas import tpu as pltpu
from jax.experimental.pallas import tpu_sc as plsc

_SCALE = 10.0
_NC = 2
_NS = 16
_CH = 128
_NB1 = 3
_NBUF = 4


@functools.lru_cache(maxsize=None)
def _make_relayout(V, D):
  dh = D // _NC
  n_full = V // _CH
  tail = V - n_full * _CH
  t0 = n_full * _CH
  bpt = (n_full + _NS - 1) // _NS
  mesh = plsc.VectorSubcoreMesh(core_axis_name="c", subcore_axis_name="s")

  @functools.partial(
      pl.kernel,
      mesh=mesh,
      out_type=jax.ShapeDtypeStruct(((_NC * t0 + 2 * tail) * dh // _CH, _CH), jnp.float32),
      scratch_types=[
          [pltpu.VMEM((dh, _CH), jnp.float32)] * _NB1,
          [pltpu.VMEM((dh, _CH), jnp.float32)] * _NB1,
          pltpu.VMEM((dh, _CH), jnp.float32),
          [pltpu.SemaphoreType.DMA] * _NB1,
          [pltpu.SemaphoreType.DMA] * _NB1,
      ],
      compiler_params=pltpu.CompilerParams(
          use_tc_tiling_on_sc=True, needs_layout_passes=False),
  )
  def relayout(wt_hbm, tail_hbm, scr_hbm, tb, rb, tt, t_sems, r_sems):
    core = lax.axis_index("c")
    tid = lax.axis_index("s")
    lanes = lax.iota(jnp.int32, 16)
    dbase = core * dh
    srow0 = core * (t0 * dh // _CH)

    def read(b, si):
      for j in range(_NC):
        pltpu.async_copy(
            wt_hbm.at[pl.ds(dbase + j * 8, 8), pl.ds(b * _CH, _CH)],
            tb[si].at[pl.ds(j * 8, 8)], t_sems[si])

    def wait_read(si):
      for j in range(_NC):
        pltpu.make_async_copy(
            wt_hbm.at[pl.ds(dbase + j * 8, 8), pl.ds(0, _CH)],
            tb[si].at[pl.ds(j * 8, 8)], t_sems[si]).wait()

    def write(b, si):
      pltpu.async_copy(
          rb[si], scr_hbm.at[pl.ds(srow0 + b * dh, dh)], r_sems[si])

    def wait_write(si):
      pltpu.make_async_copy(
          rb[si], scr_hbm.at[pl.ds(srow0, dh)], r_sems[si]).wait()

    lo = tid * bpt
    n_my = jnp.maximum(jnp.minimum(lo + bpt, n_full) - lo, 0)

    for si in range(_NB1):
      @pl.when(si < n_my)
      def _():
        read(lo + si, si)

    i0 = lax.shift_right_logical(lanes, 3)
    i1 = lanes & 7
    zeros = jnp.zeros((16,), jnp.int32)

    i0s = [lax.shift_right_logical(lanes, 3) + 2 * g for g in range(8)]
    h1 = lax.shift_left(lanes & 7, 4)

    def transpose_block(si):
      @plsc.parallel_loop(0, dh, unroll=2)
      def _(f):
        i1 = h1 + f
        for g in range(8):
          v = tb[si][f, pl.ds(g * 16, 16)]
          plsc.store_scatter(rb[si], [i0s[g], i1], v)

    def outer(k0, carry):
      for si in range(_NB1):
        k = k0 + si
        @pl.when(k < n_my)
        def _():
          b = lo + k
          wait_read(si)
          @pl.when(k >= _NB1)
          def _():
            wait_write(si)
          transpose_block(si)
          write(b, si)
          @pl.when(k + _NB1 < n_my)
          def _():
            read(b + _NB1, si)
      return carry

    lax.fori_loop(0, (bpt + _NB1 - 1) // _NB1,
                  lambda i, cr: outer(i * _NB1, cr), 0)

    for si in range(_NB1):
      @pl.when(n_my > si)
      def _():
        wait_write(si)

    if tail:
      @pl.when((tid == 0) & (core == 0))
      def _():
        pltpu.sync_copy(tail_hbm, tt)
        pltpu.sync_copy(
            tt, scr_hbm.at[pl.ds(_NC * t0 * dh // _CH, 2 * tail * dh // _CH)])

  return relayout


@functools.lru_cache(maxsize=None)
def _make_lookup(V, D, S0, S1):
  B = S0 * S1
  n_st = S0 // _CH
  n_dt = D // 8
  dh = D // _NC
  n_ch_tot = B // _CH
  n_ch = n_ch_tot // _NS
  n_tiles = n_ch_tot * n_dt
  assert n_ch % _NBUF == 0
  mesh = plsc.VectorSubcoreMesh(core_axis_name="c", subcore_axis_name="s")

  @functools.partial(
      pl.kernel,
      mesh=mesh,
      out_type=jax.ShapeDtypeStruct((n_tiles, 8, _CH), jnp.float32),
      scratch_types=[
          pltpu.VMEM((n_ch, _CH), jnp.int32),
          [pltpu.VMEM((_CH,), jnp.int32)] * _NBUF,
          [pltpu.VMEM((_CH, dh), jnp.float32)] * _NBUF,
          [pltpu.VMEM((_NC, 8, _CH), jnp.float32)] * _NBUF,
          [pltpu.SemaphoreType.DMA] * _NBUF,
          [pltpu.SemaphoreType.DMA] * _NBUF,
      ],
      compiler_params=pltpu.CompilerParams(
          use_tc_tiling_on_sc=False, needs_layout_passes=False),
  )
  def lookup(scr_hbm, idx_hbm, out_hbm, idx_v, p_v, g_b, o_b,
             in_sems, out_sems):
    core = lax.axis_index("c")
    tid = lax.axis_index("s")
    lanes = lax.iota(jnp.int32, 16)
    t0 = (V // _CH) * _CH
    tail = V - t0
    off_lo = core * t0
    off_hi = t0 + core * tail
    ch_base = tid * n_ch
    pltpu.sync_copy(idx_hbm.at[pl.ds(ch_base, n_ch)], idx_v)

    def gather(cl, bi):
      @plsc.parallel_loop(0, _CH // 16, unroll=4)
      def _(g):
        rv = idx_v[cl, pl.ds(g * 16, 16)]
        p_v[bi][pl.ds(g * 16, 16)] = rv + jnp.where(rv < t0, off_lo, off_hi)
      return pltpu.async_copy(scr_hbm.at[p_v[bi]], g_b[bi], in_sems[bi])

    def wait_gather(bi):
      pltpu.make_async_copy(scr_hbm.at[p_v[bi]], g_b[bi], in_sems[bi]).wait()

    def out_tile_base(cl):
      c = ch_base + cl
      s1 = lax.div(c, n_st)
      st = lax.rem(c, n_st)
      return (s1 * n_dt + core * _NC) * n_st + st

    def write(cl, bi):
      base = out_tile_base(cl)
      for j in range(_NC):
        pltpu.async_copy(
            o_b[bi].at[pl.ds(j, 1)],
            out_hbm.at[pl.ds(base + j * n_st, 1)], out_sems[bi])

    def wait_write(cl, bi):
      base = out_tile_base(cl)
      for j in range(_NC):
        pltpu.make_async_copy(
            o_b[bi].at[pl.ds(j, 1)],
            out_hbm.at[pl.ds(base + j * n_st, 1)], out_sems[bi]).wait()

    for bi in range(_NBUF):
      gather(bi, bi)

    def outer(c0, carry):
      for bi in range(_NBUF):
        cl = c0 + bi
        wait_gather(bi)
        @pl.when(cl >= _NBUF)
        def _():
          wait_write(cl - _NBUF, bi)

        @plsc.parallel_loop(0, _CH // 16, unroll=2)
        def _(g):
          rows = g * 16 + lanes
          for d in range(dh):
            v = plsc.load_gather(g_b[bi], [rows, jnp.full((16,), d, jnp.int32)])
            o_b[bi][d // 8, d % 8, pl.ds(g * 16, 16)] = v * _SCALE

        write(cl, bi)
        @pl.when(cl + _NBUF < n_ch)
        def _():
          gather(cl + _NBUF, bi)
      return carry

    lax.fori_loop(0, n_ch // _NBUF, lambda i, cr: outer(i * _NBUF, cr), 0)

    for bi in range(_NBUF):
      wait_write(n_ch - _NBUF + bi, bi)

  return lookup


def kernel(x, weight):
  S0, S1 = x.shape
  V, D = weight.shape
  B = S0 * S1
  dh = D // _NC
  t0 = (V // _CH) * _CH
  idx = x.T.reshape(B // _CH, _CH).astype(jnp.int32)
  tail_rows = jnp.concatenate(
      [weight[t0:, :dh], weight[t0:, dh:]], axis=0).reshape(dh, _CH)
  scratch = _make_relayout(V, D)(weight.T, tail_rows)
  scratch = scratch.reshape(-1, dh)
  out_t = _make_lookup(V, D, S0, S1)(scratch, idx)
  n_st = S0 // _CH
  n_dt = D // 8
  out = out_t.reshape(S1, n_dt, n_st, 8, _CH).transpose(2, 4, 0, 1, 3)
  return out.reshape(S0, S1, D)


# speedup vs baseline: 2.0810x; 1.1108x over previous
import functools

import jax
import jax.numpy as jnp
from jax import lax
from jax.experimental import pallas as pl
from jax.experimental.pallas import tpu as pltpu
from jax.experimental.pallas import tpu_sc as plsc

_SCALE = 10.0
_NC = 2
_NS = 16
_CH = 128
_NB1 = 3
_NBUF = 4


@functools.lru_cache(maxsize=None)
def _make_relayout(V, D):
  dh = D // _NC
  n_full = V // _CH
  tail = V - n_full * _CH
  t0 = n_full * _CH
  bpt = (n_full + _NS - 1) // _NS
  mesh = plsc.VectorSubcoreMesh(core_axis_name="c", subcore_axis_name="s")

  @functools.partial(
      pl.kernel,
      mesh=mesh,
      out_type=jax.ShapeDtypeStruct(((_NC * t0 + 2 * tail) * dh // _CH, _CH), jnp.float32),
      scratch_types=[
          [pltpu.VMEM((dh, _CH), jnp.float32)] * _NB1,
          [pltpu.VMEM((dh, _CH), jnp.float32)] * _NB1,
          pltpu.VMEM((dh, _CH), jnp.float32),
          [pltpu.SemaphoreType.DMA] * _NB1,
          [pltpu.SemaphoreType.DMA] * _NB1,
      ],
      compiler_params=pltpu.CompilerParams(
          use_tc_tiling_on_sc=True, needs_layout_passes=False),
  )
  def relayout(wt_hbm, tail_hbm, scr_hbm, tb, rb, tt, t_sems, r_sems):
    core = lax.axis_index("c")
    tid = lax.axis_index("s")
    lanes = lax.iota(jnp.int32, 16)
    dbase = core * dh
    srow0 = core * (t0 * dh // _CH)

    def read(b, si):
      for j in range(_NC):
        pltpu.async_copy(
            wt_hbm.at[pl.ds(dbase + j * 8, 8), pl.ds(b * _CH, _CH)],
            tb[si].at[pl.ds(j * 8, 8)], t_sems[si])

    def wait_read(si):
      for j in range(_NC):
        pltpu.make_async_copy(
            wt_hbm.at[pl.ds(dbase + j * 8, 8), pl.ds(0, _CH)],
            tb[si].at[pl.ds(j * 8, 8)], t_sems[si]).wait()

    def write(b, si):
      pltpu.async_copy(
          rb[si], scr_hbm.at[pl.ds(srow0 + b * dh, dh)], r_sems[si])

    def wait_write(si):
      pltpu.make_async_copy(
          rb[si], scr_hbm.at[pl.ds(srow0, dh)], r_sems[si]).wait()

    lo = tid * bpt
    n_my = jnp.maximum(jnp.minimum(lo + bpt, n_full) - lo, 0)

    for si in range(_NB1):
      @pl.when(si < n_my)
      def _():
        read(lo + si, si)

    i0 = lax.shift_right_logical(lanes, 3)
    i1 = lanes & 7
    zeros = jnp.zeros((16,), jnp.int32)

    i0s = [lax.shift_right_logical(lanes, 3) + 2 * g for g in range(8)]
    h1 = lax.shift_left(lanes & 7, 4)

    def transpose_block(si):
      @plsc.parallel_loop(0, dh, unroll=2)
      def _(f):
        i1 = h1 + f
        for g in range(8):
          v = tb[si][f, pl.ds(g * 16, 16)]
          plsc.store_scatter(rb[si], [i0s[g], i1], v)

    def outer(k0, carry):
      for si in range(_NB1):
        k = k0 + si
        @pl.when(k < n_my)
        def _():
          b = lo + k
          wait_read(si)
          @pl.when(k >= _NB1)
          def _():
            wait_write(si)
          transpose_block(si)
          write(b, si)
          @pl.when(k + _NB1 < n_my)
          def _():
            read(b + _NB1, si)
      return carry

    lax.fori_loop(0, (bpt + _NB1 - 1) // _NB1,
                  lambda i, cr: outer(i * _NB1, cr), 0)

    for si in range(_NB1):
      @pl.when(n_my > si)
      def _():
        wait_write(si)

    if tail:
      @pl.when((tid == 0) & (core == 0))
      def _():
        pltpu.sync_copy(tail_hbm, tt)
        pltpu.sync_copy(
            tt, scr_hbm.at[pl.ds(_NC * t0 * dh // _CH, 2 * tail * dh // _CH)])

  return relayout


@functools.lru_cache(maxsize=None)
def _make_lookup(V, D, S0, S1):
  B = S0 * S1
  n_st = S0 // _CH
  n_dt = D // 8
  dh = D // _NC
  n_ch_tot = B // _CH
  n_ch = n_ch_tot // _NS
  n_tiles = n_ch_tot * n_dt
  assert n_ch % _NBUF == 0
  mesh = plsc.VectorSubcoreMesh(core_axis_name="c", subcore_axis_name="s")

  @functools.partial(
      pl.kernel,
      mesh=mesh,
      out_type=jax.ShapeDtypeStruct((n_tiles, 8, _CH), jnp.float32),
      scratch_types=[
          pltpu.VMEM((n_ch, _CH), jnp.int32),
          [pltpu.VMEM((_CH,), jnp.int32)] * _NBUF,
          [pltpu.VMEM((_CH, dh), jnp.float32)] * _NBUF,
          [pltpu.VMEM((_NC, 8, _CH), jnp.float32)] * _NBUF,
          [pltpu.SemaphoreType.DMA] * _NBUF,
          [pltpu.SemaphoreType.DMA] * _NBUF,
      ],
      compiler_params=pltpu.CompilerParams(
          use_tc_tiling_on_sc=False, needs_layout_passes=False),
  )
  def lookup(scr_hbm, idx_hbm, out_hbm, idx_v, p_v, g_b, o_b,
             in_sems, out_sems):
    core = lax.axis_index("c")
    tid = lax.axis_index("s")
    lanes = lax.iota(jnp.int32, 16)
    zeros = jnp.zeros((16,), jnp.int32)
    rows_g = [g * 16 + lanes for g in range(8)]
    t0 = (V // _CH) * _CH
    tail = V - t0
    off_lo = core * t0
    off_hi = t0 + core * tail
    ch_base = tid * n_ch
    pltpu.sync_copy(idx_hbm.at[pl.ds(ch_base, n_ch)], idx_v)

    def gather(cl, bi):
      @plsc.parallel_loop(0, _CH // 16, unroll=4)
      def _(g):
        rv = idx_v[cl, pl.ds(g * 16, 16)]
        p_v[bi][pl.ds(g * 16, 16)] = rv + jnp.where(rv < t0, off_lo, off_hi)
      return pltpu.async_copy(scr_hbm.at[p_v[bi]], g_b[bi], in_sems[bi])

    def wait_gather(bi):
      pltpu.make_async_copy(scr_hbm.at[p_v[bi]], g_b[bi], in_sems[bi]).wait()

    def out_tile_base(cl):
      c = ch_base + cl
      s1 = lax.div(c, n_st)
      st = lax.rem(c, n_st)
      return (s1 * n_dt + core * _NC) * n_st + st

    def write(cl, bi):
      base = out_tile_base(cl)
      for j in range(_NC):
        pltpu.async_copy(
            o_b[bi].at[pl.ds(j, 1)],
            out_hbm.at[pl.ds(base + j * n_st, 1)], out_sems[bi])

    def wait_write(cl, bi):
      base = out_tile_base(cl)
      for j in range(_NC):
        pltpu.make_async_copy(
            o_b[bi].at[pl.ds(j, 1)],
            out_hbm.at[pl.ds(base + j * n_st, 1)], out_sems[bi]).wait()

    for bi in range(_NBUF):
      gather(bi, bi)

    def outer(c0, carry):
      for bi in range(_NBUF):
        cl = c0 + bi
        wait_gather(bi)
        @pl.when(cl >= _NBUF)
        def _():
          wait_write(cl - _NBUF, bi)

        @plsc.parallel_loop(0, dh, unroll=2)
        def _(d):
          cd = zeros + d
          j = lax.shift_right_logical(d, 3)
          d8 = d & 7
          for g in range(8):
            v = plsc.load_gather(g_b[bi], [rows_g[g], cd])
            o_b[bi][j, d8, pl.ds(g * 16, 16)] = v * _SCALE

        write(cl, bi)
        @pl.when(cl + _NBUF < n_ch)
        def _():
          gather(cl + _NBUF, bi)
      return carry

    lax.fori_loop(0, n_ch // _NBUF, lambda i, cr: outer(i * _NBUF, cr), 0)

    for bi in range(_NBUF):
      wait_write(n_ch - _NBUF + bi, bi)

  return lookup


def kernel(x, weight):
  S0, S1 = x.shape
  V, D = weight.shape
  B = S0 * S1
  dh = D // _NC
  t0 = (V // _CH) * _CH
  idx = x.T.reshape(B // _CH, _CH).astype(jnp.int32)
  tail_rows = jnp.concatenate(
      [weight[t0:, :dh], weight[t0:, dh:]], axis=0).reshape(dh, _CH)
  scratch = _make_relayout(V, D)(weight.T, tail_rows)
  scratch = scratch.reshape(-1, dh)
  out_t = _make_lookup(V, D, S0, S1)(scratch, idx)
  n_st = S0 // _CH
  n_dt = D // 8
  out = out_t.reshape(S1, n_dt, n_st, 8, _CH).transpose(2, 4, 0, 1, 3)
  return out.reshape(S0, S1, D)


# speedup vs baseline: 2.4701x; 1.1870x over previous
import functools

import jax
import jax.numpy as jnp
from jax import lax
from jax.experimental import pallas as pl
from jax.experimental.pallas import tpu as pltpu
from jax.experimental.pallas import tpu_sc as plsc

_SCALE = 10.0
_NC = 2
_NS = 16
_CH = 128
_NB1 = 4
_NBUF = 8


@functools.lru_cache(maxsize=None)
def _make_relayout(V, D):
  dh = D // _NC
  n_full = V // _CH
  tail = V - n_full * _CH
  t0 = n_full * _CH
  bpt = (n_full + _NS - 1) // _NS
  mesh = plsc.VectorSubcoreMesh(core_axis_name="c", subcore_axis_name="s")

  @functools.partial(
      pl.kernel,
      mesh=mesh,
      out_type=jax.ShapeDtypeStruct(((_NC * t0 + 2 * tail) * dh // _CH, _CH), jnp.float32),
      scratch_types=[
          [pltpu.VMEM((dh, _CH), jnp.float32)] * _NB1,
          [pltpu.VMEM((dh, _CH), jnp.float32)] * _NB1,
          pltpu.VMEM((dh, _CH), jnp.float32),
          [pltpu.SemaphoreType.DMA] * _NB1,
          [pltpu.SemaphoreType.DMA] * _NB1,
      ],
      compiler_params=pltpu.CompilerParams(
          use_tc_tiling_on_sc=True, needs_layout_passes=False),
  )
  def relayout(wt_hbm, tail_hbm, scr_hbm, tb, rb, tt, t_sems, r_sems):
    core = lax.axis_index("c")
    tid = lax.axis_index("s")
    lanes = lax.iota(jnp.int32, 16)
    dbase = core * dh
    srow0 = core * (t0 * dh // _CH)

    def read(b, si):
      for j in range(_NC):
        pltpu.async_copy(
            wt_hbm.at[pl.ds(dbase + j * 8, 8), pl.ds(b * _CH, _CH)],
            tb[si].at[pl.ds(j * 8, 8)], t_sems[si])

    def wait_read(si):
      for j in range(_NC):
        pltpu.make_async_copy(
            wt_hbm.at[pl.ds(dbase + j * 8, 8), pl.ds(0, _CH)],
            tb[si].at[pl.ds(j * 8, 8)], t_sems[si]).wait()

    def write(b, si):
      pltpu.async_copy(
          rb[si], scr_hbm.at[pl.ds(srow0 + b * dh, dh)], r_sems[si])

    def wait_write(si):
      pltpu.make_async_copy(
          rb[si], scr_hbm.at[pl.ds(srow0, dh)], r_sems[si]).wait()

    lo = tid * bpt
    n_my = jnp.maximum(jnp.minimum(lo + bpt, n_full) - lo, 0)

    for si in range(_NB1):
      @pl.when(si < n_my)
      def _():
        read(lo + si, si)

    i0 = lax.shift_right_logical(lanes, 3)
    i1 = lanes & 7
    zeros = jnp.zeros((16,), jnp.int32)

    i0s = [lax.shift_right_logical(lanes, 3) + 2 * g for g in range(8)]
    h1 = lax.shift_left(lanes & 7, 4)

    def transpose_block(si):
      @plsc.parallel_loop(0, dh, unroll=2)
      def _(f):
        i1 = h1 + f
        for g in range(8):
          v = tb[si][f, pl.ds(g * 16, 16)]
          plsc.store_scatter(rb[si], [i0s[g], i1], v)

    def outer(k0, carry):
      for si in range(_NB1):
        k = k0 + si
        @pl.when(k < n_my)
        def _():
          b = lo + k
          wait_read(si)
          @pl.when(k >= _NB1)
          def _():
            wait_write(si)
          transpose_block(si)
          write(b, si)
          @pl.when(k + _NB1 < n_my)
          def _():
            read(b + _NB1, si)
      return carry

    lax.fori_loop(0, (bpt + _NB1 - 1) // _NB1,
                  lambda i, cr: outer(i * _NB1, cr), 0)

    for si in range(_NB1):
      @pl.when(n_my > si)
      def _():
        wait_write(si)

    if tail:
      @pl.when((tid == 0) & (core == 0))
      def _():
        pltpu.sync_copy(tail_hbm, tt)
        pltpu.sync_copy(
            tt, scr_hbm.at[pl.ds(_NC * t0 * dh // _CH, 2 * tail * dh // _CH)])

  return relayout


@functools.lru_cache(maxsize=None)
def _make_lookup(V, D, S0, S1):
  B = S0 * S1
  n_st = S0 // _CH
  n_dt = D // 8
  dh = D // _NC
  n_ch_tot = B // _CH
  n_ch = n_ch_tot // _NS
  n_tiles = n_ch_tot * n_dt
  assert n_ch % _NBUF == 0
  mesh = plsc.VectorSubcoreMesh(core_axis_name="c", subcore_axis_name="s")

  @functools.partial(
      pl.kernel,
      mesh=mesh,
      out_type=jax.ShapeDtypeStruct((n_tiles, 8, _CH), jnp.float32),
      scratch_types=[
          pltpu.VMEM((n_ch, _CH), jnp.int32),
          [pltpu.VMEM((_CH,), jnp.int32)] * _NBUF,
          [pltpu.VMEM((_CH, dh), jnp.float32)] * _NBUF,
          [pltpu.VMEM((_NC, 8, _CH), jnp.float32)] * _NBUF,
          [pltpu.SemaphoreType.DMA] * _NBUF,
          [pltpu.SemaphoreType.DMA] * _NBUF,
      ],
      compiler_params=pltpu.CompilerParams(
          use_tc_tiling_on_sc=False, needs_layout_passes=False),
  )
  def lookup(scr_hbm, idx_hbm, out_hbm, idx_v, p_v, g_b, o_b,
             in_sems, out_sems):
    core = lax.axis_index("c")
    tid = lax.axis_index("s")
    lanes = lax.iota(jnp.int32, 16)
    zeros = jnp.zeros((16,), jnp.int32)
    rows_g = [g * 16 + lanes for g in range(8)]
    t0 = (V // _CH) * _CH
    tail = V - t0
    off_lo = core * t0
    off_hi = t0 + core * tail
    ch_base = tid * n_ch
    pltpu.sync_copy(idx_hbm.at[pl.ds(ch_base, n_ch)], idx_v)

    def gather(cl, bi):
      @plsc.parallel_loop(0, _CH // 16, unroll=4)
      def _(g):
        rv = idx_v[cl, pl.ds(g * 16, 16)]
        p_v[bi][pl.ds(g * 16, 16)] = rv + jnp.where(rv < t0, off_lo, off_hi)
      return pltpu.async_copy(scr_hbm.at[p_v[bi]], g_b[bi], in_sems[bi])

    def wait_gather(bi):
      pltpu.make_async_copy(scr_hbm.at[p_v[bi]], g_b[bi], in_sems[bi]).wait()

    def out_tile_base(cl):
      c = ch_base + cl
      s1 = lax.div(c, n_st)
      st = lax.rem(c, n_st)
      return (s1 * n_dt + core * _NC) * n_st + st

    def write(cl, bi):
      base = out_tile_base(cl)
      for j in range(_NC):
        pltpu.async_copy(
            o_b[bi].at[pl.ds(j, 1)],
            out_hbm.at[pl.ds(base + j * n_st, 1)], out_sems[bi])

    def wait_write(cl, bi):
      base = out_tile_base(cl)
      for j in range(_NC):
        pltpu.make_async_copy(
            o_b[bi].at[pl.ds(j, 1)],
            out_hbm.at[pl.ds(base + j * n_st, 1)], out_sems[bi]).wait()

    for bi in range(_NBUF):
      gather(bi, bi)

    def outer(c0, carry):
      for bi in range(_NBUF):
        cl = c0 + bi
        wait_gather(bi)
        @pl.when(cl >= _NBUF)
        def _():
          wait_write(cl - _NBUF, bi)

        @plsc.parallel_loop(0, dh, unroll=2)
        def _(d):
          cd = zeros + d
          j = lax.shift_right_logical(d, 3)
          d8 = d & 7
          for g in range(8):
            v = plsc.load_gather(g_b[bi], [rows_g[g], cd])
            o_b[bi][j, d8, pl.ds(g * 16, 16)] = v * _SCALE

        write(cl, bi)
        @pl.when(cl + _NBUF < n_ch)
        def _():
          gather(cl + _NBUF, bi)
      return carry

    lax.fori_loop(0, n_ch // _NBUF, lambda i, cr: outer(i * _NBUF, cr), 0)

    for bi in range(_NBUF):
      wait_write(n_ch - _NBUF + bi, bi)

  return lookup


def kernel(x, weight):
  S0, S1 = x.shape
  V, D = weight.shape
  B = S0 * S1
  dh = D // _NC
  t0 = (V // _CH) * _CH
  idx = x.T.reshape(B // _CH, _CH).astype(jnp.int32)
  tail_rows = jnp.concatenate(
      [weight[t0:, :dh], weight[t0:, dh:]], axis=0).reshape(dh, _CH)
  scratch = _make_relayout(V, D)(weight.T, tail_rows)
  scratch = scratch.reshape(-1, dh)
  out_t = _make_lookup(V, D, S0, S1)(scratch, idx)
  n_st = S0 // _CH
  n_dt = D // 8
  out = out_t.reshape(S1, n_dt, n_st, 8, _CH).transpose(2, 4, 0, 1, 3)
  return out.reshape(S0, S1, D)


# speedup vs baseline: 2.7139x; 1.0987x over previous
import functools

import jax
import jax.numpy as jnp
from jax import lax
from jax.experimental import pallas as pl
from jax.experimental.pallas import tpu as pltpu
from jax.experimental.pallas import tpu_sc as plsc

_SCALE = 10.0
_NC = 2
_NS = 16
_CH = 128
_NB1 = 6
_NBUF = 10


@functools.lru_cache(maxsize=None)
def _make_relayout(V, D):
  dh = D // _NC
  n_full = V // _CH
  tail = V - n_full * _CH
  t0 = n_full * _CH
  bpt = (n_full + _NS - 1) // _NS
  mesh = plsc.VectorSubcoreMesh(core_axis_name="c", subcore_axis_name="s")

  @functools.partial(
      pl.kernel,
      mesh=mesh,
      out_type=jax.ShapeDtypeStruct(((_NC * t0 + 2 * tail) * dh // _CH, _CH), jnp.float32),
      scratch_types=[
          [pltpu.VMEM((dh, _CH), jnp.float32)] * _NB1,
          [pltpu.VMEM((dh, _CH), jnp.float32)] * _NB1,
          pltpu.VMEM((dh, _CH), jnp.float32),
          [pltpu.SemaphoreType.DMA] * _NB1,
          [pltpu.SemaphoreType.DMA] * _NB1,
      ],
      compiler_params=pltpu.CompilerParams(
          use_tc_tiling_on_sc=True, needs_layout_passes=False),
  )
  def relayout(wt_hbm, tail_hbm, scr_hbm, tb, rb, tt, t_sems, r_sems):
    core = lax.axis_index("c")
    tid = lax.axis_index("s")
    lanes = lax.iota(jnp.int32, 16)
    dbase = core * dh
    srow0 = core * (t0 * dh // _CH)

    def read(b, si):
      for j in range(_NC):
        pltpu.async_copy(
            wt_hbm.at[pl.ds(dbase + j * 8, 8), pl.ds(b * _CH, _CH)],
            tb[si].at[pl.ds(j * 8, 8)], t_sems[si])

    def wait_read(si):
      for j in range(_NC):
        pltpu.make_async_copy(
            wt_hbm.at[pl.ds(dbase + j * 8, 8), pl.ds(0, _CH)],
            tb[si].at[pl.ds(j * 8, 8)], t_sems[si]).wait()

    def write(b, si):
      pltpu.async_copy(
          rb[si], scr_hbm.at[pl.ds(srow0 + b * dh, dh)], r_sems[si])

    def wait_write(si):
      pltpu.make_async_copy(
          rb[si], scr_hbm.at[pl.ds(srow0, dh)], r_sems[si]).wait()

    lo = tid * bpt
    n_my = jnp.maximum(jnp.minimum(lo + bpt, n_full) - lo, 0)

    for si in range(_NB1):
      @pl.when(si < n_my)
      def _():
        read(lo + si, si)

    i0 = lax.shift_right_logical(lanes, 3)
    i1 = lanes & 7
    zeros = jnp.zeros((16,), jnp.int32)

    i0s = [lax.shift_right_logical(lanes, 3) + 2 * g for g in range(8)]
    h1 = lax.shift_left(lanes & 7, 4)

    def transpose_block(si):
      @plsc.parallel_loop(0, dh, unroll=2)
      def _(f):
        i1 = h1 + f
        for g in range(8):
          v = tb[si][f, pl.ds(g * 16, 16)]
          plsc.store_scatter(rb[si], [i0s[g], i1], v)

    def outer(k0, carry):
      for si in range(_NB1):
        k = k0 + si
        @pl.when(k < n_my)
        def _():
          b = lo + k
          wait_read(si)
          @pl.when(k >= _NB1)
          def _():
            wait_write(si)
          transpose_block(si)
          write(b, si)
          @pl.when(k + _NB1 < n_my)
          def _():
            read(b + _NB1, si)
      return carry

    lax.fori_loop(0, (bpt + _NB1 - 1) // _NB1,
                  lambda i, cr: outer(i * _NB1, cr), 0)

    for si in range(_NB1):
      @pl.when(n_my > si)
      def _():
        wait_write(si)

    if tail:
      @pl.when((tid == 0) & (core == 0))
      def _():
        pltpu.sync_copy(tail_hbm, tt)
        pltpu.sync_copy(
            tt, scr_hbm.at[pl.ds(_NC * t0 * dh // _CH, 2 * tail * dh // _CH)])

  return relayout


@functools.lru_cache(maxsize=None)
def _make_lookup(V, D, S0, S1):
  B = S0 * S1
  n_st = S0 // _CH
  n_dt = D // 8
  dh = D // _NC
  n_ch_tot = B // _CH
  n_ch = n_ch_tot // _NS
  n_tiles = n_ch_tot * n_dt
  assert n_ch % _NBUF == 0
  mesh = plsc.VectorSubcoreMesh(core_axis_name="c", subcore_axis_name="s")

  @functools.partial(
      pl.kernel,
      mesh=mesh,
      out_type=jax.ShapeDtypeStruct((n_tiles, 8, _CH), jnp.float32),
      scratch_types=[
          pltpu.VMEM((n_ch, _CH), jnp.int32),
          [pltpu.VMEM((_CH,), jnp.int32)] * _NBUF,
          [pltpu.VMEM((_CH, dh), jnp.float32)] * _NBUF,
          [pltpu.VMEM((_NC, 8, _CH), jnp.float32)] * _NBUF,
          [pltpu.SemaphoreType.DMA] * _NBUF,
          [pltpu.SemaphoreType.DMA] * _NBUF,
      ],
      compiler_params=pltpu.CompilerParams(
          use_tc_tiling_on_sc=False, needs_layout_passes=False),
  )
  def lookup(scr_hbm, idx_hbm, out_hbm, idx_v, p_v, g_b, o_b,
             in_sems, out_sems):
    core = lax.axis_index("c")
    tid = lax.axis_index("s")
    lanes = lax.iota(jnp.int32, 16)
    zeros = jnp.zeros((16,), jnp.int32)
    rows_g = [g * 16 + lanes for g in range(8)]
    t0 = (V // _CH) * _CH
    tail = V - t0
    off_lo = core * t0
    off_hi = t0 + core * tail
    ch_base = tid * n_ch
    pltpu.sync_copy(idx_hbm.at[pl.ds(ch_base, n_ch)], idx_v)

    def gather(cl, bi):
      @plsc.parallel_loop(0, _CH // 16, unroll=4)
      def _(g):
        rv = idx_v[cl, pl.ds(g * 16, 16)]
        p_v[bi][pl.ds(g * 16, 16)] = rv + jnp.where(rv < t0, off_lo, off_hi)
      return pltpu.async_copy(scr_hbm.at[p_v[bi]], g_b[bi], in_sems[bi])

    def wait_gather(bi):
      pltpu.make_async_copy(scr_hbm.at[p_v[bi]], g_b[bi], in_sems[bi]).wait()

    def out_tile_base(cl):
      c = ch_base + cl
      s1 = lax.div(c, n_st)
      st = lax.rem(c, n_st)
      return (s1 * n_dt + core * _NC) * n_st + st

    def write(cl, bi):
      base = out_tile_base(cl)
      for j in range(_NC):
        pltpu.async_copy(
            o_b[bi].at[pl.ds(j, 1)],
            out_hbm.at[pl.ds(base + j * n_st, 1)], out_sems[bi])

    def wait_write(cl, bi):
      base = out_tile_base(cl)
      for j in range(_NC):
        pltpu.make_async_copy(
            o_b[bi].at[pl.ds(j, 1)],
            out_hbm.at[pl.ds(base + j * n_st, 1)], out_sems[bi]).wait()

    for bi in range(_NBUF):
      gather(bi, bi)

    def outer(c0, carry):
      for bi in range(_NBUF):
        cl = c0 + bi
        wait_gather(bi)
        @pl.when(cl >= _NBUF)
        def _():
          wait_write(cl - _NBUF, bi)

        @plsc.parallel_loop(0, dh, unroll=2)
        def _(d):
          cd = zeros + d
          j = lax.shift_right_logical(d, 3)
          d8 = d & 7
          for g in range(8):
            v = plsc.load_gather(g_b[bi], [rows_g[g], cd])
            o_b[bi][j, d8, pl.ds(g * 16, 16)] = v * _SCALE

        write(cl, bi)
        @pl.when(cl + _NBUF < n_ch)
        def _():
          gather(cl + _NBUF, bi)
      return carry

    lax.fori_loop(0, n_ch // _NBUF, lambda i, cr: outer(i * _NBUF, cr), 0)

    for bi in range(_NBUF):
      wait_write(n_ch - _NBUF + bi, bi)

  return lookup


def kernel(x, weight):
  S0, S1 = x.shape
  V, D = weight.shape
  B = S0 * S1
  dh = D // _NC
  t0 = (V // _CH) * _CH
  idx = x.T.reshape(B // _CH, _CH).astype(jnp.int32)
  tail_rows = jnp.concatenate(
      [weight[t0:, :dh], weight[t0:, dh:]], axis=0).reshape(dh, _CH)
  scratch = _make_relayout(V, D)(weight.T, tail_rows)
  scratch = scratch.reshape(-1, dh)
  out_t = _make_lookup(V, D, S0, S1)(scratch, idx)
  n_st = S0 // _CH
  n_dt = D // 8
  out = out_t.reshape(S1, n_dt, n_st, 8, _CH).transpose(2, 4, 0, 1, 3)
  return out.reshape(S0, S1, D)
